# baseline (speedup 1.0000x reference)
import functools

import jax
import jax.numpy as jnp
from jax import lax
from jax.experimental import pallas as pl
from jax.experimental.pallas import tpu as pltpu
from jax.experimental.pallas import tpu_sc as plsc

F32 = jnp.float32
BF16 = jnp.bfloat16
LAT = 128
N_DOWN_NODES = 4000
NC = 2
NS = 16
NW = NC * NS

_BN = 2000


def _full(a):
    n = a.ndim
    return pl.BlockSpec(a.shape, lambda i: (0,) * n)


def _row_spec(bn, k):
    return pl.BlockSpec((bn, k), lambda i: (i, 0))


def _ln(y, g, o):
    m = jnp.mean(y, axis=-1, keepdims=True)
    v = jnp.mean(jnp.square(y - m), axis=-1, keepdims=True)
    return (y - m) * lax.rsqrt(v + 1e-5) * g + o


def _swish(x):
    return x * jax.nn.sigmoid(x)


def _mlp_body(x_ref, w1_ref, b1_ref, w2_ref, b2_ref, g_ref, o_ref, out_ref,
              *, residual):
    x = x_ref[...]
    h = _swish(jnp.dot(x, w1_ref[...], preferred_element_type=F32) + b1_ref[...])
    y = jnp.dot(h, w2_ref[...], preferred_element_type=F32) + b2_ref[...]
    y = _ln(y, g_ref[...], o_ref[...])
    if residual:
        y = y + x
    out_ref[...] = y


def _mlp_block(x, mlp, ln, residual):
    (w1, b1), (w2, b2) = mlp
    g, o = ln
    n, k = x.shape
    d = w2.shape[1]
    args = (x, w1, b1.reshape(1, -1), w2, b2.reshape(1, -1),
            g.reshape(1, -1), o.reshape(1, -1))
    return pl.pallas_call(
        functools.partial(_mlp_body, residual=residual),
        grid=(n // _BN,),
        in_specs=[_row_spec(_BN, k)] + [_full(a) for a in args[1:]],
        out_specs=_row_spec(_BN, d),
        out_shape=jax.ShapeDtypeStruct((n, d), F32),
    )(*args)


def _edge_body(x_ref, m_ref, w1_ref, b1_ref, w2_ref, b2_ref,
               g_ref, o_ref, out_ref):
    x = x_ref[...]
    h = jnp.dot(x, w1_ref[...], preferred_element_type=F32)
    h = _swish(h + m_ref[...] + b1_ref[...])
    y = jnp.dot(h, w2_ref[...], preferred_element_type=F32) + b2_ref[...]
    out_ref[...] = x + _ln(y, g_ref[...], o_ref[...])


def _edge_block(e, m, w1e, b1, w2, b2, ln):
    g, o = ln
    n = e.shape[0]
    args = (e, m, w1e, b1.reshape(1, -1), w2, b2.reshape(1, -1),
            g.reshape(1, -1), o.reshape(1, -1))
    return pl.pallas_call(
        _edge_body,
        grid=(n // _BN,),
        in_specs=[_row_spec(_BN, LAT)] * 2 + [_full(a) for a in args[2:]],
        out_specs=_row_spec(_BN, LAT),
        out_shape=jax.ShapeDtypeStruct((n, LAT), F32),
    )(*args)


def _node_down_body(x_ref, a0_ref, a1_ref, u0_ref, u1_ref, wd_ref, wi_ref,
                    wu_ref, b1_ref, w2_ref, b2_ref, g_ref, o_ref, out_ref):
    x = x_ref[...]
    ai = a0_ref[...] + a1_ref[...]
    au = u0_ref[...] + u1_ref[...]
    h = (jnp.dot(x, wd_ref[...], preferred_element_type=F32)
         + jnp.dot(ai, wi_ref[...], preferred_element_type=F32)
         + jnp.dot(au, wu_ref[...], preferred_element_type=F32)
         + b1_ref[...])
    h = _swish(h)
    y = jnp.dot(h, w2_ref[...], preferred_element_type=F32) + b2_ref[...]
    out_ref[...] = x + _ln(y, g_ref[...], o_ref[...])


def _node_down_block(x, aggi, aggu, w1, b1, w2, b2, ln):
    g, o = ln
    n = x.shape[0]
    args = (x, aggi[0], aggi[1], aggu[0], aggu[1],
            w1[0:LAT], w1[LAT:2 * LAT], w1[2 * LAT:3 * LAT],
            b1.reshape(1, -1), w2, b2.reshape(1, -1),
            g.reshape(1, -1), o.reshape(1, -1))
    return pl.pallas_call(
        _node_down_body,
        grid=(n // _BN,),
        in_specs=[_row_spec(_BN, LAT)] * 5 + [_full(a) for a in args[5:]],
        out_specs=_row_spec(_BN, LAT),
        out_shape=jax.ShapeDtypeStruct((n, LAT), F32),
    )(*args)


def _proj3_body(x_ref, wa_ref, wb_ref, wc_ref, oa_ref, ob_ref, oc_ref):
    x = x_ref[...]
    oa_ref[...] = jnp.dot(x, wa_ref[...], preferred_element_type=F32)
    ob_ref[...] = jnp.dot(x, wb_ref[...], preferred_element_type=F32)
    oc_ref[...] = jnp.dot(x, wc_ref[...], preferred_element_type=F32)


def _proj3(x, wa, wb, wc):
    n = x.shape[0]
    sh = jax.ShapeDtypeStruct((n, LAT), F32)
    return pl.pallas_call(
        _proj3_body,
        grid=(n // _BN,),
        in_specs=[_row_spec(_BN, LAT), _full(wa), _full(wb), _full(wc)],
        out_specs=[_row_spec(_BN, LAT)] * 3,
        out_shape=[sh, sh, sh],
    )(x, wa, wb, wc)


def _proj1_body(x_ref, w_ref, o_ref):
    o_ref[...] = jnp.dot(x_ref[...], w_ref[...], preferred_element_type=F32)


def _proj1(x, w):
    n = x.shape[0]
    return pl.pallas_call(
        _proj1_body,
        grid=(n // _BN,),
        in_specs=[_row_spec(_BN, LAT), _full(w)],
        out_specs=_row_spec(_BN, LAT),
        out_shape=jax.ShapeDtypeStruct((n, LAT), F32),
    )(x, w)


def _dec_body(x_ref, w1_ref, b1_ref, w2_ref, b2_ref, out_ref):
    x = x_ref[...]
    h = _swish(jnp.dot(x, w1_ref[...], preferred_element_type=F32) + b1_ref[...])
    out_ref[...] = jnp.dot(h, w2_ref[...], preferred_element_type=F32) + b2_ref[...]


def _dec_block(x, w1, b1, w2, b2):
    n = x.shape[0]
    w2p = jnp.zeros((LAT, LAT), F32).at[:, :w2.shape[1]].set(w2)
    b2p = jnp.zeros((1, LAT), F32).at[:, :w2.shape[1]].set(b2)
    args = (x, w1, b1.reshape(1, -1), w2p, b2p)
    out = pl.pallas_call(
        _dec_body,
        grid=(n // _BN,),
        in_specs=[_row_spec(_BN, LAT)] + [_full(a) for a in args[1:]],
        out_specs=_row_spec(_BN, LAT),
        out_shape=jax.ShapeDtypeStruct((n, LAT), F32),
    )(*args)
    return out[:, 0]


def _sc_gather_add(tab_a, tab_b, idx_a, idx_b, chunk):
    e = idx_a.shape[0]
    per_w = e // NW
    nch = per_w // chunk
    assert per_w % chunk == 0 and chunk % 8 == 0 and chunk <= 128
    mesh = plsc.VectorSubcoreMesh(core_axis_name="c", subcore_axis_name="s", num_cores=NC, num_subcores=NS)

    depth = 4
    nq, rem = divmod(nch, depth)

    def body(ta, tb, ia, ib, o, ia_v, ib_v, rs, sas, sbs, sws):
        wid = lax.axis_index("s") * NC + lax.axis_index("c")
        base = wid * per_w
        pltpu.sync_copy(ia.at[pl.ds(base, per_w)], ia_v)
        pltpu.sync_copy(ib.at[pl.ds(base, per_w)], ib_v)

        def run_group(j0, width):
            offs = [j0 + t * chunk for t in range(width)]
            gas = [pltpu.async_copy(ta.at[ia_v.at[pl.ds(offs[t], chunk)]],
                                    rs[t], sas[t]) for t in range(width)]
            gbs = []
            for t in range(width):
                gas[t].wait()
                gbs.append(pltpu.async_copy(
                    tb.at[ib_v.at[pl.ds(offs[t], chunk)]], rs[t], sbs[t],
                    add=True))
            ws = []
            for t in range(width):
                gbs[t].wait()
                ws.append(pltpu.async_copy(
                    rs[t], o.at[pl.ds(base + offs[t], chunk)], sws[t]))
            for w in ws:
                w.wait()

        @pl.loop(0, nq)
        def _quad(k):
            run_group(depth * k * chunk, depth)

        if rem:
            run_group(nq * depth * chunk, rem)

    f = pl.kernel(
        body,
        out_type=jax.ShapeDtypeStruct((e, LAT), F32),
        mesh=mesh,
        scratch_types=[
            pltpu.VMEM((per_w,), jnp.int32),
            pltpu.VMEM((per_w,), jnp.int32),
            [pltpu.VMEM((chunk, LAT), F32)] * depth,
            [pltpu.SemaphoreType.DMA] * depth,
            [pltpu.SemaphoreType.DMA] * depth,
            [pltpu.SemaphoreType.DMA] * depth,
        ],
    )
    return f(tab_a, tab_b, idx_a, idx_b)


def _sc_segsum(vals, idx, init, chunk):
    e = idx.shape[0]
    per_w = e // NW
    nch = per_w // chunk
    assert per_w % chunk == 0 and chunk % 8 == 0 and chunk <= 128
    rows_per_tile = 400
    mesh = plsc.VectorSubcoreMesh(core_axis_name="c", subcore_axis_name="s", num_cores=NC, num_subcores=NS)

    npairs, odd = divmod(nch, 2)

    def body(vals_h, idx_h, init_h, out_h, idx_v, val0, val1, acc_sh,
             sl0, sl1, ss0, ss1):
        cid = lax.axis_index("c")
        sid = lax.axis_index("s")
        wid = sid * NC + cid

        @pl.when(sid < N_DOWN_NODES // rows_per_tile)
        def _init():
            r0 = sid * rows_per_tile
            pltpu.sync_copy(init_h.at[cid, pl.ds(r0, rows_per_tile)],
                            acc_sh.at[pl.ds(r0, rows_per_tile)])

        base = wid * per_w
        pltpu.sync_copy(idx_h.at[pl.ds(base, per_w)], idx_v)
        plsc.subcore_barrier()

        @pl.loop(0, npairs)
        def _pair(k):
            j0 = 2 * k * chunk
            j1 = j0 + chunk
            l0 = pltpu.async_copy(vals_h.at[pl.ds(base + j0, chunk)], val0, sl0)
            l1 = pltpu.async_copy(vals_h.at[pl.ds(base + j1, chunk)], val1, sl1)
            l0.wait()
            s0 = pltpu.async_copy(val0, acc_sh.at[idx_v.at[pl.ds(j0, chunk)]],
                                  ss0, add=True)
            l1.wait()
            s1 = pltpu.async_copy(val1, acc_sh.at[idx_v.at[pl.ds(j1, chunk)]],
                                  ss1, add=True)
            s0.wait()
            s1.wait()

        if odd:
            jl = 2 * npairs * chunk
            pltpu.sync_copy(vals_h.at[pl.ds(base + jl, chunk)], val0)
            pltpu.sync_copy(val0, acc_sh.at[idx_v.at[pl.ds(jl, chunk)]],
                            add=True)

        plsc.subcore_barrier()

        @pl.when(sid < N_DOWN_NODES // rows_per_tile)
        def _writeout():
            r0 = sid * rows_per_tile
            pltpu.sync_copy(acc_sh.at[pl.ds(r0, rows_per_tile)],
                            out_h.at[cid, pl.ds(r0, rows_per_tile)])

    f = pl.kernel(
        body,
        out_type=jax.ShapeDtypeStruct((NC, N_DOWN_NODES, LAT), F32),
        mesh=mesh,
        scratch_types=[
            pltpu.VMEM((per_w,), jnp.int32),
            pltpu.VMEM((chunk, LAT), F32),
            pltpu.VMEM((chunk, LAT), F32),
            pltpu.VMEM_SHARED((N_DOWN_NODES, LAT), F32),
            pltpu.SemaphoreType.DMA,
            pltpu.SemaphoreType.DMA,
            pltpu.SemaphoreType.DMA,
            pltpu.SemaphoreType.DMA,
        ],
    )
    return f(vals, idx, init)


def kernel(upstream_x, downstream_x, intra_edge_index, u2d_src, u2d_dst,
           intra_edge_attr, u2d_edge_attr, params):
    p = params
    up = _mlp_block(upstream_x, p["enc_up"]["mlp"], p["enc_up"]["ln"], False)
    down = _mlp_block(downstream_x, p["enc_down"]["mlp"], p["enc_down"]["ln"], False)
    ihalf = intra_edge_index.shape[1] // 2
    isrc = (intra_edge_index[0, :ihalf], intra_edge_index[0, ihalf:])
    idst = (intra_edge_index[1, :ihalf], intra_edge_index[1, ihalf:])
    e_i = [
        _mlp_block(intra_edge_attr[:ihalf], p["emb_intra"]["mlp"],
                   p["emb_intra"]["ln"], False),
        _mlp_block(intra_edge_attr[ihalf:], p["emb_intra"]["mlp"],
                   p["emb_intra"]["ln"], False),
    ]

    half = u2d_src.shape[0] // 2
    usrc = (u2d_src[:half], u2d_src[half:])
    udst = (u2d_dst[:half], u2d_dst[half:])
    e_u = [
        _mlp_block(u2d_edge_attr[:half], p["emb_u2d"]["mlp"],
                   p["emb_u2d"]["ln"], False),
        _mlp_block(u2d_edge_attr[half:], p["emb_u2d"]["mlp"],
                   p["emb_u2d"]["ln"], False),
    ]

    zeros = jnp.zeros((NC, N_DOWN_NODES, LAT), F32)

    for sp in p["steps"]:
        (w1i, b1i), (w2i, b2i) = sp["edge_intra"]["mlp"]
        (w1u, b1u), (w2u, b2u) = sp["edge_u2d"]["mlp"]

        pis, pid, pud = _proj3(down, w1i[LAT:2 * LAT], w1i[2 * LAT:],
                               w1u[2 * LAT:])
        pus = _proj1(up, w1u[LAT:2 * LAT])

        mi0 = _sc_gather_add(pis, pid, isrc[0], idst[0], chunk=80)
        mu0 = _sc_gather_add(pus, pud, usrc[0], udst[0], chunk=120)
        mi1 = _sc_gather_add(pis, pid, isrc[1], idst[1], chunk=80)
        mu1 = _sc_gather_add(pus, pud, usrc[1], udst[1], chunk=120)

        e_i[0] = _edge_block(e_i[0], mi0, w1i[:LAT], b1i, w2i, b2i,
                             sp["edge_intra"]["ln"])
        e_u[0] = _edge_block(e_u[0], mu0, w1u[:LAT], b1u, w2u, b2u,
                             sp["edge_u2d"]["ln"])
        e_i[1] = _edge_block(e_i[1], mi1, w1i[:LAT], b1i, w2i, b2i,
                             sp["edge_intra"]["ln"])
        e_u[1] = _edge_block(e_u[1], mu1, w1u[:LAT], b1u, w2u, b2u,
                             sp["edge_u2d"]["ln"])

        parti = _sc_segsum(e_i[0], idst[0], zeros, chunk=80)
        partu = _sc_segsum(e_u[0], udst[0], zeros, chunk=120)
        aggi = _sc_segsum(e_i[1], idst[1], parti, chunk=80)
        aggu = _sc_segsum(e_u[1], udst[1], partu, chunk=120)

        (w1d, b1d), (w2d, b2d) = sp["node_down"]["mlp"]
        down = _node_down_block(down, aggi, aggu, w1d, b1d, w2d, b2d,
                                sp["node_down"]["ln"])
        up = _mlp_block(up, sp["node_up"]["mlp"], sp["node_up"]["ln"], True)

    (wd1, bd1), (wd2, bd2) = p["dec"]["mlp"]
    return _dec_block(down, wd1, bd1, wd2, bd2)


# speedup vs baseline: 1.0226x; 1.0226x over previous
import functools

import jax
import jax.numpy as jnp
from jax import lax
from jax.experimental import pallas as pl
from jax.experimental.pallas import tpu as pltpu
from jax.experimental.pallas import tpu_sc as plsc

F32 = jnp.float32
BF16 = jnp.bfloat16
LAT = 128
N_DOWN_NODES = 4000
NC = 2
NS = 16
NW = NC * NS

_BN = 2000


def _full(a):
    n = a.ndim
    return pl.BlockSpec(a.shape, lambda i: (0,) * n)


def _row_spec(bn, k):
    return pl.BlockSpec((bn, k), lambda i: (i, 0))


def _ln(y, g, o):
    m = jnp.mean(y, axis=-1, keepdims=True)
    v = jnp.mean(jnp.square(y - m), axis=-1, keepdims=True)
    return (y - m) * lax.rsqrt(v + 1e-5) * g + o


def _swish(x):
    return x * jax.nn.sigmoid(x)


def _mlp_body(x_ref, w1_ref, b1_ref, w2_ref, b2_ref, g_ref, o_ref, out_ref,
              *, residual):
    x = x_ref[...]
    h = _swish(jnp.dot(x, w1_ref[...], preferred_element_type=F32) + b1_ref[...])
    y = jnp.dot(h, w2_ref[...], preferred_element_type=F32) + b2_ref[...]
    y = _ln(y, g_ref[...], o_ref[...])
    if residual:
        y = y + x
    out_ref[...] = y


def _mlp_block(x, mlp, ln, residual):
    (w1, b1), (w2, b2) = mlp
    g, o = ln
    n, k = x.shape
    d = w2.shape[1]
    args = (x, w1, b1.reshape(1, -1), w2, b2.reshape(1, -1),
            g.reshape(1, -1), o.reshape(1, -1))
    return pl.pallas_call(
        functools.partial(_mlp_body, residual=residual),
        grid=(n // _BN,),
        in_specs=[_row_spec(_BN, k)] + [_full(a) for a in args[1:]],
        out_specs=_row_spec(_BN, d),
        out_shape=jax.ShapeDtypeStruct((n, d), F32),
    )(*args)


def _edge_body(x_ref, m_ref, w1_ref, b1_ref, w2_ref, b2_ref,
               g_ref, o_ref, out_ref):
    x = x_ref[...]
    h = jnp.dot(x, w1_ref[...], preferred_element_type=F32)
    h = _swish(h + m_ref[...] + b1_ref[...])
    y = jnp.dot(h, w2_ref[...], preferred_element_type=F32) + b2_ref[...]
    out_ref[...] = x + _ln(y, g_ref[...], o_ref[...])


def _edge_block(e, m, w1e, b1, w2, b2, ln):
    g, o = ln
    n = e.shape[0]
    args = (e, m, w1e, b1.reshape(1, -1), w2, b2.reshape(1, -1),
            g.reshape(1, -1), o.reshape(1, -1))
    return pl.pallas_call(
        _edge_body,
        grid=(n // _BN,),
        in_specs=[_row_spec(_BN, LAT)] * 2 + [_full(a) for a in args[2:]],
        out_specs=_row_spec(_BN, LAT),
        out_shape=jax.ShapeDtypeStruct((n, LAT), F32),
    )(*args)


def _node_down_body(x_ref, a0_ref, a1_ref, u0_ref, u1_ref, wd_ref, wi_ref,
                    wu_ref, b1_ref, w2_ref, b2_ref, g_ref, o_ref, out_ref):
    x = x_ref[...]
    ai = a0_ref[...] + a1_ref[...]
    au = u0_ref[...] + u1_ref[...]
    h = (jnp.dot(x, wd_ref[...], preferred_element_type=F32)
         + jnp.dot(ai, wi_ref[...], preferred_element_type=F32)
         + jnp.dot(au, wu_ref[...], preferred_element_type=F32)
         + b1_ref[...])
    h = _swish(h)
    y = jnp.dot(h, w2_ref[...], preferred_element_type=F32) + b2_ref[...]
    out_ref[...] = x + _ln(y, g_ref[...], o_ref[...])


def _node_down_block(x, aggi, aggu, w1, b1, w2, b2, ln):
    g, o = ln
    n = x.shape[0]
    args = (x, aggi[0], aggi[1], aggu[0], aggu[1],
            w1[0:LAT], w1[LAT:2 * LAT], w1[2 * LAT:3 * LAT],
            b1.reshape(1, -1), w2, b2.reshape(1, -1),
            g.reshape(1, -1), o.reshape(1, -1))
    return pl.pallas_call(
        _node_down_body,
        grid=(n // _BN,),
        in_specs=[_row_spec(_BN, LAT)] * 5 + [_full(a) for a in args[5:]],
        out_specs=_row_spec(_BN, LAT),
        out_shape=jax.ShapeDtypeStruct((n, LAT), F32),
    )(*args)


def _proj3_body(x_ref, wa_ref, wb_ref, wc_ref, oa_ref, ob_ref, oc_ref):
    x = x_ref[...]
    oa_ref[...] = jnp.dot(x, wa_ref[...], preferred_element_type=F32)
    ob_ref[...] = jnp.dot(x, wb_ref[...], preferred_element_type=F32)
    oc_ref[...] = jnp.dot(x, wc_ref[...], preferred_element_type=F32)


def _proj3(x, wa, wb, wc):
    n = x.shape[0]
    sh = jax.ShapeDtypeStruct((n, LAT), F32)
    return pl.pallas_call(
        _proj3_body,
        grid=(n // _BN,),
        in_specs=[_row_spec(_BN, LAT), _full(wa), _full(wb), _full(wc)],
        out_specs=[_row_spec(_BN, LAT)] * 3,
        out_shape=[sh, sh, sh],
    )(x, wa, wb, wc)


def _proj1_body(x_ref, w_ref, o_ref):
    o_ref[...] = jnp.dot(x_ref[...], w_ref[...], preferred_element_type=F32)


def _proj1(x, w):
    n = x.shape[0]
    return pl.pallas_call(
        _proj1_body,
        grid=(n // _BN,),
        in_specs=[_row_spec(_BN, LAT), _full(w)],
        out_specs=_row_spec(_BN, LAT),
        out_shape=jax.ShapeDtypeStruct((n, LAT), F32),
    )(x, w)


def _dec_body(x_ref, w1_ref, b1_ref, w2_ref, b2_ref, out_ref):
    x = x_ref[...]
    h = _swish(jnp.dot(x, w1_ref[...], preferred_element_type=F32) + b1_ref[...])
    out_ref[...] = jnp.dot(h, w2_ref[...], preferred_element_type=F32) + b2_ref[...]


def _dec_block(x, w1, b1, w2, b2):
    n = x.shape[0]
    w2p = jnp.zeros((LAT, LAT), F32).at[:, :w2.shape[1]].set(w2)
    b2p = jnp.zeros((1, LAT), F32).at[:, :w2.shape[1]].set(b2)
    args = (x, w1, b1.reshape(1, -1), w2p, b2p)
    out = pl.pallas_call(
        _dec_body,
        grid=(n // _BN,),
        in_specs=[_row_spec(_BN, LAT)] + [_full(a) for a in args[1:]],
        out_specs=_row_spec(_BN, LAT),
        out_shape=jax.ShapeDtypeStruct((n, LAT), F32),
    )(*args)
    return out[:, 0]


def _sc_gather_add(tab_a, tab_b, idx_a, idx_b, chunk):
    e = idx_a.shape[0]
    per_w = e // NW
    nch = per_w // chunk
    assert per_w % chunk == 0 and chunk % 8 == 0 and chunk <= 128
    mesh = plsc.VectorSubcoreMesh(core_axis_name="c", subcore_axis_name="s", num_cores=NC, num_subcores=NS)

    depth = 4
    nq, rem = divmod(nch, depth)

    def body(ta, tb, ia, ib, o, ia_v, ib_v, rs, sas, sbs, sws):
        wid = lax.axis_index("s") * NC + lax.axis_index("c")
        base = wid * per_w
        pltpu.sync_copy(ia.at[pl.ds(base, per_w)], ia_v)
        pltpu.sync_copy(ib.at[pl.ds(base, per_w)], ib_v)

        def run_group(j0, width):
            offs = [j0 + t * chunk for t in range(width)]
            gas = [pltpu.async_copy(ta.at[ia_v.at[pl.ds(offs[t], chunk)]],
                                    rs[t], sas[t]) for t in range(width)]
            gbs = []
            for t in range(width):
                gas[t].wait()
                gbs.append(pltpu.async_copy(
                    tb.at[ib_v.at[pl.ds(offs[t], chunk)]], rs[t], sbs[t],
                    add=True))
            ws = []
            for t in range(width):
                gbs[t].wait()
                ws.append(pltpu.async_copy(
                    rs[t], o.at[pl.ds(base + offs[t], chunk)], sws[t]))
            for w in ws:
                w.wait()

        @pl.loop(0, nq)
        def _quad(k):
            run_group(depth * k * chunk, depth)

        if rem:
            run_group(nq * depth * chunk, rem)

    f = pl.kernel(
        body,
        out_type=jax.ShapeDtypeStruct((e, LAT), F32),
        mesh=mesh,
        scratch_types=[
            pltpu.VMEM((per_w,), jnp.int32),
            pltpu.VMEM((per_w,), jnp.int32),
            [pltpu.VMEM((chunk, LAT), F32)] * depth,
            [pltpu.SemaphoreType.DMA] * depth,
            [pltpu.SemaphoreType.DMA] * depth,
            [pltpu.SemaphoreType.DMA] * depth,
        ],
    )
    return f(tab_a, tab_b, idx_a, idx_b)


def _sc_segsum(vals, idx, init, chunk):
    e = idx.shape[0]
    per_w = e // NW
    nch = per_w // chunk
    assert per_w % chunk == 0 and chunk % 8 == 0 and chunk <= 128
    rows_per_tile = 400
    mesh = plsc.VectorSubcoreMesh(core_axis_name="c", subcore_axis_name="s", num_cores=NC, num_subcores=NS)

    npairs, odd = divmod(nch, 2)

    def body(vals_h, idx_h, init_h, out_h, idx_v, val0, val1, acc_sh,
             sl0, sl1, ss0, ss1):
        cid = lax.axis_index("c")
        sid = lax.axis_index("s")
        wid = sid * NC + cid

        @pl.when(sid < N_DOWN_NODES // rows_per_tile)
        def _init():
            r0 = sid * rows_per_tile
            pltpu.sync_copy(init_h.at[cid, pl.ds(r0, rows_per_tile)],
                            acc_sh.at[pl.ds(r0, rows_per_tile)])

        base = wid * per_w
        pltpu.sync_copy(idx_h.at[pl.ds(base, per_w)], idx_v)
        plsc.subcore_barrier()

        @pl.loop(0, npairs)
        def _pair(k):
            j0 = 2 * k * chunk
            j1 = j0 + chunk
            l0 = pltpu.async_copy(vals_h.at[pl.ds(base + j0, chunk)], val0, sl0)
            l1 = pltpu.async_copy(vals_h.at[pl.ds(base + j1, chunk)], val1, sl1)
            l0.wait()
            s0 = pltpu.async_copy(val0, acc_sh.at[idx_v.at[pl.ds(j0, chunk)]],
                                  ss0, add=True)
            l1.wait()
            s1 = pltpu.async_copy(val1, acc_sh.at[idx_v.at[pl.ds(j1, chunk)]],
                                  ss1, add=True)
            s0.wait()
            s1.wait()

        if odd:
            jl = 2 * npairs * chunk
            pltpu.sync_copy(vals_h.at[pl.ds(base + jl, chunk)], val0)
            pltpu.sync_copy(val0, acc_sh.at[idx_v.at[pl.ds(jl, chunk)]],
                            add=True)

        plsc.subcore_barrier()

        @pl.when(sid < N_DOWN_NODES // rows_per_tile)
        def _writeout():
            r0 = sid * rows_per_tile
            pltpu.sync_copy(acc_sh.at[pl.ds(r0, rows_per_tile)],
                            out_h.at[cid, pl.ds(r0, rows_per_tile)])

    f = pl.kernel(
        body,
        out_type=jax.ShapeDtypeStruct((NC, N_DOWN_NODES, LAT), F32),
        mesh=mesh,
        scratch_types=[
            pltpu.VMEM((per_w,), jnp.int32),
            pltpu.VMEM((chunk, LAT), F32),
            pltpu.VMEM((chunk, LAT), F32),
            pltpu.VMEM_SHARED((N_DOWN_NODES, LAT), F32),
            pltpu.SemaphoreType.DMA,
            pltpu.SemaphoreType.DMA,
            pltpu.SemaphoreType.DMA,
            pltpu.SemaphoreType.DMA,
        ],
    )
    return f(vals, idx, init)


def kernel(upstream_x, downstream_x, intra_edge_index, u2d_src, u2d_dst,
           intra_edge_attr, u2d_edge_attr, params):
    p = params
    up = _mlp_block(upstream_x, p["enc_up"]["mlp"], p["enc_up"]["ln"], False)
    down = _mlp_block(downstream_x, p["enc_down"]["mlp"], p["enc_down"]["ln"], False)
    e_i = _mlp_block(intra_edge_attr, p["emb_intra"]["mlp"],
                     p["emb_intra"]["ln"], False)

    half = u2d_src.shape[0] // 2
    usrc = (u2d_src[:half], u2d_src[half:])
    udst = (u2d_dst[:half], u2d_dst[half:])
    e_u = [
        _mlp_block(u2d_edge_attr[:half], p["emb_u2d"]["mlp"],
                   p["emb_u2d"]["ln"], False),
        _mlp_block(u2d_edge_attr[half:], p["emb_u2d"]["mlp"],
                   p["emb_u2d"]["ln"], False),
    ]

    i_src = intra_edge_index[0]
    i_dst = intra_edge_index[1]
    zeros = jnp.zeros((NC, N_DOWN_NODES, LAT), F32)

    for sp in p["steps"]:
        (w1i, b1i), (w2i, b2i) = sp["edge_intra"]["mlp"]
        (w1u, b1u), (w2u, b2u) = sp["edge_u2d"]["mlp"]

        pis, pid, pud = _proj3(down, w1i[LAT:2 * LAT], w1i[2 * LAT:],
                               w1u[2 * LAT:])
        pus = _proj1(up, w1u[LAT:2 * LAT])

        mi = _sc_gather_add(pis, pid, i_src, i_dst, chunk=80)
        mu0 = _sc_gather_add(pus, pud, usrc[0], udst[0], chunk=120)
        mu1 = _sc_gather_add(pus, pud, usrc[1], udst[1], chunk=120)

        e_i = _edge_block(e_i, mi, w1i[:LAT], b1i, w2i, b2i,
                          sp["edge_intra"]["ln"])
        e_u[0] = _edge_block(e_u[0], mu0, w1u[:LAT], b1u, w2u, b2u,
                             sp["edge_u2d"]["ln"])
        e_u[1] = _edge_block(e_u[1], mu1, w1u[:LAT], b1u, w2u, b2u,
                             sp["edge_u2d"]["ln"])

        aggi = _sc_segsum(e_i, i_dst, zeros, chunk=80)
        part = _sc_segsum(e_u[0], udst[0], zeros, chunk=120)
        aggu = _sc_segsum(e_u[1], udst[1], part, chunk=120)

        (w1d, b1d), (w2d, b2d) = sp["node_down"]["mlp"]
        down = _node_down_block(down, aggi, aggu, w1d, b1d, w2d, b2d,
                                sp["node_down"]["ln"])
        up = _mlp_block(up, sp["node_up"]["mlp"], sp["node_up"]["ln"], True)

    (wd1, bd1), (wd2, bd2) = p["dec"]["mlp"]
    return _dec_block(down, wd1, bd1, wd2, bd2)


# speedup vs baseline: 1.0303x; 1.0076x over previous
import functools

import jax
import jax.numpy as jnp
from jax import lax
from jax.experimental import pallas as pl
from jax.experimental.pallas import tpu as pltpu
from jax.experimental.pallas import tpu_sc as plsc

F32 = jnp.float32
BF16 = jnp.bfloat16
LAT = 128
N_DOWN_NODES = 4000
NC = 2
NS = 16
NW = NC * NS

_BN = 2000


def _full(a):
    n = a.ndim
    return pl.BlockSpec(a.shape, lambda i: (0,) * n)


def _row_spec(bn, k):
    return pl.BlockSpec((bn, k), lambda i: (i, 0))


def _ln(y, g, o):
    m = jnp.mean(y, axis=-1, keepdims=True)
    v = jnp.mean(jnp.square(y - m), axis=-1, keepdims=True)
    return (y - m) * lax.rsqrt(v + 1e-5) * g + o


def _swish(x):
    return x * jax.nn.sigmoid(x)


def _mlp_body(x_ref, w1_ref, b1_ref, w2_ref, b2_ref, g_ref, o_ref, out_ref,
              *, residual):
    x = x_ref[...]
    h = _swish(jnp.dot(x, w1_ref[...], preferred_element_type=F32) + b1_ref[...])
    y = jnp.dot(h, w2_ref[...], preferred_element_type=F32) + b2_ref[...]
    y = _ln(y, g_ref[...], o_ref[...])
    if residual:
        y = y + x
    out_ref[...] = y


def _mlp_block(x, mlp, ln, residual, rows=None, row_off=0):
    (w1, b1), (w2, b2) = mlp
    g, o = ln
    n, k = x.shape
    if rows is None:
        rows = n
    ob = row_off // _BN
    assert row_off % _BN == 0 and rows % _BN == 0
    d = w2.shape[1]
    args = (x, w1, b1.reshape(1, -1), w2, b2.reshape(1, -1),
            g.reshape(1, -1), o.reshape(1, -1))
    return pl.pallas_call(
        functools.partial(_mlp_body, residual=residual),
        grid=(rows // _BN,),
        in_specs=[pl.BlockSpec((_BN, k), lambda i: (i + ob, 0))]
        + [_full(a) for a in args[1:]],
        out_specs=_row_spec(_BN, d),
        out_shape=jax.ShapeDtypeStruct((rows, d), F32),
    )(*args)


def _edge_body(x_ref, m_ref, w1_ref, b1_ref, w2_ref, b2_ref,
               g_ref, o_ref, out_ref):
    x = x_ref[...]
    h = jnp.dot(x, w1_ref[...], preferred_element_type=F32)
    h = _swish(h + m_ref[...] + b1_ref[...])
    y = jnp.dot(h, w2_ref[...], preferred_element_type=F32) + b2_ref[...]
    out_ref[...] = x + _ln(y, g_ref[...], o_ref[...])


def _edge_block(e, m, w1e, b1, w2, b2, ln):
    g, o = ln
    n = e.shape[0]
    args = (e, m, w1e, b1.reshape(1, -1), w2, b2.reshape(1, -1),
            g.reshape(1, -1), o.reshape(1, -1))
    return pl.pallas_call(
        _edge_body,
        grid=(n // _BN,),
        in_specs=[_row_spec(_BN, LAT)] * 2 + [_full(a) for a in args[2:]],
        out_specs=_row_spec(_BN, LAT),
        out_shape=jax.ShapeDtypeStruct((n, LAT), F32),
    )(*args)


def _node_down_body(x_ref, a0_ref, a1_ref, u0_ref, u1_ref, wd_ref, wi_ref,
                    wu_ref, b1_ref, w2_ref, b2_ref, g_ref, o_ref, out_ref):
    x = x_ref[...]
    ai = a0_ref[...] + a1_ref[...]
    au = u0_ref[...] + u1_ref[...]
    h = (jnp.dot(x, wd_ref[...], preferred_element_type=F32)
         + jnp.dot(ai, wi_ref[...], preferred_element_type=F32)
         + jnp.dot(au, wu_ref[...], preferred_element_type=F32)
         + b1_ref[...])
    h = _swish(h)
    y = jnp.dot(h, w2_ref[...], preferred_element_type=F32) + b2_ref[...]
    out_ref[...] = x + _ln(y, g_ref[...], o_ref[...])


def _node_down_block(x, aggi, aggu, w1, b1, w2, b2, ln):
    g, o = ln
    n = x.shape[0]
    args = (x, aggi[0], aggi[1], aggu[0], aggu[1],
            w1[0:LAT], w1[LAT:2 * LAT], w1[2 * LAT:3 * LAT],
            b1.reshape(1, -1), w2, b2.reshape(1, -1),
            g.reshape(1, -1), o.reshape(1, -1))
    return pl.pallas_call(
        _node_down_body,
        grid=(n // _BN,),
        in_specs=[_row_spec(_BN, LAT)] * 5 + [_full(a) for a in args[5:]],
        out_specs=_row_spec(_BN, LAT),
        out_shape=jax.ShapeDtypeStruct((n, LAT), F32),
    )(*args)


def _proj3_body(x_ref, wa_ref, wb_ref, wc_ref, oa_ref, ob_ref, oc_ref):
    x = x_ref[...]
    oa_ref[...] = jnp.dot(x, wa_ref[...], preferred_element_type=F32)
    ob_ref[...] = jnp.dot(x, wb_ref[...], preferred_element_type=F32)
    oc_ref[...] = jnp.dot(x, wc_ref[...], preferred_element_type=F32)


def _proj3(x, wa, wb, wc):
    n = x.shape[0]
    sh = jax.ShapeDtypeStruct((n, LAT), F32)
    return pl.pallas_call(
        _proj3_body,
        grid=(n // _BN,),
        in_specs=[_row_spec(_BN, LAT), _full(wa), _full(wb), _full(wc)],
        out_specs=[_row_spec(_BN, LAT)] * 3,
        out_shape=[sh, sh, sh],
    )(x, wa, wb, wc)


def _proj1_body(x_ref, w_ref, o_ref):
    o_ref[...] = jnp.dot(x_ref[...], w_ref[...], preferred_element_type=F32)


def _proj1(x, w):
    n = x.shape[0]
    return pl.pallas_call(
        _proj1_body,
        grid=(n // _BN,),
        in_specs=[_row_spec(_BN, LAT), _full(w)],
        out_specs=_row_spec(_BN, LAT),
        out_shape=jax.ShapeDtypeStruct((n, LAT), F32),
    )(x, w)


def _dec_body(x_ref, w1_ref, b1_ref, w2_ref, b2_ref, out_ref):
    x = x_ref[...]
    h = _swish(jnp.dot(x, w1_ref[...], preferred_element_type=F32) + b1_ref[...])
    out_ref[...] = jnp.dot(h, w2_ref[...], preferred_element_type=F32) + b2_ref[...]


def _dec_block(x, w1, b1, w2, b2):
    n = x.shape[0]
    w2p = jnp.zeros((LAT, LAT), F32).at[:, :w2.shape[1]].set(w2)
    b2p = jnp.zeros((1, LAT), F32).at[:, :w2.shape[1]].set(b2)
    args = (x, w1, b1.reshape(1, -1), w2p, b2p)
    out = pl.pallas_call(
        _dec_body,
        grid=(n // _BN,),
        in_specs=[_row_spec(_BN, LAT)] + [_full(a) for a in args[1:]],
        out_specs=_row_spec(_BN, LAT),
        out_shape=jax.ShapeDtypeStruct((n, LAT), F32),
    )(*args)
    return out[:, 0]


def _sc_gather_add(tab_a, tab_b, idx_a, idx_b, chunk):
    e = idx_a.shape[0]
    per_w = e // NW
    nch = per_w // chunk
    assert per_w % chunk == 0 and chunk % 8 == 0 and chunk <= 128
    mesh = plsc.VectorSubcoreMesh(core_axis_name="c", subcore_axis_name="s", num_cores=NC, num_subcores=NS)

    depth = 4
    nq, rem = divmod(nch, depth)

    def body(ta, tb, ia, ib, o, ia_v, ib_v, rs, sas, sbs, sws):
        wid = lax.axis_index("s") * NC + lax.axis_index("c")
        base = wid * per_w
        pltpu.sync_copy(ia.at[pl.ds(base, per_w)], ia_v)
        pltpu.sync_copy(ib.at[pl.ds(base, per_w)], ib_v)

        def run_group(j0, width):
            offs = [j0 + t * chunk for t in range(width)]
            gas = [pltpu.async_copy(ta.at[ia_v.at[pl.ds(offs[t], chunk)]],
                                    rs[t], sas[t]) for t in range(width)]
            gbs = []
            for t in range(width):
                gas[t].wait()
                gbs.append(pltpu.async_copy(
                    tb.at[ib_v.at[pl.ds(offs[t], chunk)]], rs[t], sbs[t],
                    add=True))
            ws = []
            for t in range(width):
                gbs[t].wait()
                ws.append(pltpu.async_copy(
                    rs[t], o.at[pl.ds(base + offs[t], chunk)], sws[t]))
            for w in ws:
                w.wait()

        @pl.loop(0, nq)
        def _quad(k):
            run_group(depth * k * chunk, depth)

        if rem:
            run_group(nq * depth * chunk, rem)

    f = pl.kernel(
        body,
        out_type=jax.ShapeDtypeStruct((e, LAT), F32),
        mesh=mesh,
        scratch_types=[
            pltpu.VMEM((per_w,), jnp.int32),
            pltpu.VMEM((per_w,), jnp.int32),
            [pltpu.VMEM((chunk, LAT), F32)] * depth,
            [pltpu.SemaphoreType.DMA] * depth,
            [pltpu.SemaphoreType.DMA] * depth,
            [pltpu.SemaphoreType.DMA] * depth,
        ],
    )
    return f(tab_a, tab_b, idx_a, idx_b)


def _sc_segsum(vals, idx, init, chunk):
    e = idx.shape[0]
    per_w = e // NW
    nch = per_w // chunk
    assert per_w % chunk == 0 and chunk % 8 == 0 and chunk <= 128
    rows_per_tile = 400
    mesh = plsc.VectorSubcoreMesh(core_axis_name="c", subcore_axis_name="s", num_cores=NC, num_subcores=NS)

    npairs, odd = divmod(nch, 2)

    def body(vals_h, idx_h, init_h, out_h, idx_v, val0, val1, acc_sh,
             sl0, sl1, ss0, ss1):
        cid = lax.axis_index("c")
        sid = lax.axis_index("s")
        wid = sid * NC + cid

        @pl.when(sid < N_DOWN_NODES // rows_per_tile)
        def _init():
            r0 = sid * rows_per_tile
            pltpu.sync_copy(init_h.at[cid, pl.ds(r0, rows_per_tile)],
                            acc_sh.at[pl.ds(r0, rows_per_tile)])

        base = wid * per_w
        pltpu.sync_copy(idx_h.at[pl.ds(base, per_w)], idx_v)
        plsc.subcore_barrier()

        @pl.loop(0, npairs)
        def _pair(k):
            j0 = 2 * k * chunk
            j1 = j0 + chunk
            l0 = pltpu.async_copy(vals_h.at[pl.ds(base + j0, chunk)], val0, sl0)
            l1 = pltpu.async_copy(vals_h.at[pl.ds(base + j1, chunk)], val1, sl1)
            l0.wait()
            s0 = pltpu.async_copy(val0, acc_sh.at[idx_v.at[pl.ds(j0, chunk)]],
                                  ss0, add=True)
            l1.wait()
            s1 = pltpu.async_copy(val1, acc_sh.at[idx_v.at[pl.ds(j1, chunk)]],
                                  ss1, add=True)
            s0.wait()
            s1.wait()

        if odd:
            jl = 2 * npairs * chunk
            pltpu.sync_copy(vals_h.at[pl.ds(base + jl, chunk)], val0)
            pltpu.sync_copy(val0, acc_sh.at[idx_v.at[pl.ds(jl, chunk)]],
                            add=True)

        plsc.subcore_barrier()

        @pl.when(sid < N_DOWN_NODES // rows_per_tile)
        def _writeout():
            r0 = sid * rows_per_tile
            pltpu.sync_copy(acc_sh.at[pl.ds(r0, rows_per_tile)],
                            out_h.at[cid, pl.ds(r0, rows_per_tile)])

    f = pl.kernel(
        body,
        out_type=jax.ShapeDtypeStruct((NC, N_DOWN_NODES, LAT), F32),
        mesh=mesh,
        scratch_types=[
            pltpu.VMEM((per_w,), jnp.int32),
            pltpu.VMEM((chunk, LAT), F32),
            pltpu.VMEM((chunk, LAT), F32),
            pltpu.VMEM_SHARED((N_DOWN_NODES, LAT), F32),
            pltpu.SemaphoreType.DMA,
            pltpu.SemaphoreType.DMA,
            pltpu.SemaphoreType.DMA,
            pltpu.SemaphoreType.DMA,
        ],
    )
    return f(vals, idx, init)


def kernel(upstream_x, downstream_x, intra_edge_index, u2d_src, u2d_dst,
           intra_edge_attr, u2d_edge_attr, params):
    p = params
    up = _mlp_block(upstream_x, p["enc_up"]["mlp"], p["enc_up"]["ln"], False)
    down = _mlp_block(downstream_x, p["enc_down"]["mlp"], p["enc_down"]["ln"], False)
    e_i = _mlp_block(intra_edge_attr, p["emb_intra"]["mlp"],
                     p["emb_intra"]["ln"], False)

    n_u = u2d_src.shape[0]
    s0 = 2 * n_u // 3
    usrc = (u2d_src[:s0], u2d_src[s0:])
    udst = (u2d_dst[:s0], u2d_dst[s0:])
    e_u = [
        _mlp_block(u2d_edge_attr, p["emb_u2d"]["mlp"],
                   p["emb_u2d"]["ln"], False, rows=s0),
        _mlp_block(u2d_edge_attr, p["emb_u2d"]["mlp"],
                   p["emb_u2d"]["ln"], False, rows=n_u - s0, row_off=s0),
    ]

    i_src = intra_edge_index[0]
    i_dst = intra_edge_index[1]
    zeros = jnp.zeros((NC, N_DOWN_NODES, LAT), F32)

    for sp in p["steps"]:
        (w1i, b1i), (w2i, b2i) = sp["edge_intra"]["mlp"]
        (w1u, b1u), (w2u, b2u) = sp["edge_u2d"]["mlp"]

        pis, pid, pud = _proj3(down, w1i[LAT:2 * LAT], w1i[2 * LAT:],
                               w1u[2 * LAT:])
        pus = _proj1(up, w1u[LAT:2 * LAT])

        mi = _sc_gather_add(pis, pid, i_src, i_dst, chunk=80)
        mu0 = _sc_gather_add(pus, pud, usrc[0], udst[0], chunk=80)
        mu1 = _sc_gather_add(pus, pud, usrc[1], udst[1], chunk=80)

        e_i = _edge_block(e_i, mi, w1i[:LAT], b1i, w2i, b2i,
                          sp["edge_intra"]["ln"])
        e_u[0] = _edge_block(e_u[0], mu0, w1u[:LAT], b1u, w2u, b2u,
                             sp["edge_u2d"]["ln"])
        e_u[1] = _edge_block(e_u[1], mu1, w1u[:LAT], b1u, w2u, b2u,
                             sp["edge_u2d"]["ln"])

        aggi = _sc_segsum(e_i, i_dst, zeros, chunk=80)
        part = _sc_segsum(e_u[0], udst[0], zeros, chunk=80)
        aggu = _sc_segsum(e_u[1], udst[1], part, chunk=80)

        (w1d, b1d), (w2d, b2d) = sp["node_down"]["mlp"]
        down = _node_down_block(down, aggi, aggu, w1d, b1d, w2d, b2d,
                                sp["node_down"]["ln"])
        up = _mlp_block(up, sp["node_up"]["mlp"], sp["node_up"]["ln"], True)

    (wd1, bd1), (wd2, bd2) = p["dec"]["mlp"]
    return _dec_block(down, wd1, bd1, wd2, bd2)
